# baseline (speedup 1.0000x reference)
import functools

import jax
import jax.numpy as jnp
from jax import lax
from jax.experimental import pallas as pl
from jax.experimental.pallas import tpu as pltpu
from jax.experimental.pallas import tpu_sc as plsc

_VOCAB = 38462 * 26
_D = 16
_BATCH = 16384
_NF = 26

_NC = 2
_NS = 16
_NW = _NC * _NS

_CB_FULL = _VOCAB // 128
_TAIL = _VOCAB - _CB_FULL * 128
_GROWS = _CB_FULL * 16 + 16

_BT = _BATCH // 128
_BT_PER_W = _BT // _NW

_mesh = plsc.VectorSubcoreMesh(core_axis_name="c", subcore_axis_name="s")
_params = pltpu.CompilerParams(use_tc_tiling_on_sc=True,
                               needs_layout_passes=False)


def _worker_id():
    return lax.axis_index("s") * _NC + lax.axis_index("c")


@functools.partial(
    pl.kernel,
    mesh=_mesh,
    out_type=jax.ShapeDtypeStruct((_GROWS, 128), jnp.float32),
    scratch_types=[
        pltpu.VMEM((16, 128), jnp.float32),
        pltpu.VMEM((16, 128), jnp.float32),
    ],
    compiler_params=_params,
)
def _k1_pack(tableT, tailT, tblpack, tbuf, obuf):
    wid = _worker_id()
    iota = lax.iota(jnp.int32, 16)

    def pack_cols(ncol4):
        def cbody(c4, carry):
            for k in range(4):
                col = plsc.load_gather(
                    tbuf, [iota, jnp.full((16,), k, jnp.int32) + c4 * 4])
                row = c4 // 2
                woff = ((c4 % 2) * 4 + k) * 16
                obuf[row, pl.ds(woff, 16)] = col
            return carry
        lax.fori_loop(0, ncol4, cbody, 0)

    def process(cb):
        pltpu.sync_copy(tableT.at[pl.ds(0, 16), pl.ds(cb * 128, 128)], tbuf)
        pack_cols(32)
        pltpu.sync_copy(obuf, tblpack.at[pl.ds(cb * 16, 16), :])

    def tbody(t, carry):
        cb = t * _NW + wid

        @pl.when(cb < _CB_FULL)
        def _():
            process(cb)
        return carry

    lax.fori_loop(0, (_CB_FULL + _NW - 1) // _NW, tbody, 0)

    @pl.when(wid == 4)
    def _():
        pltpu.sync_copy(tailT, tbuf)
        pack_cols(32)
        pltpu.sync_copy(obuf, tblpack.at[pl.ds(_CB_FULL * 16, 16), :])


@functools.partial(
    pl.kernel,
    mesh=_mesh,
    out_type=jax.ShapeDtypeStruct((_NF, _D, _BATCH), jnp.float32),
    scratch_types=[
        pltpu.VMEM((32, 128), jnp.int32),
        pltpu.VMEM((_NF, 128), jnp.int32),
        pltpu.VMEM((_NF, 128), jnp.int32),
        pltpu.VMEM((128, 128), jnp.float32),
        pltpu.VMEM((2, 8, 128), jnp.float32),
        pltpu.SemaphoreType.DMA,
    ],
    compiler_params=_params,
)
def _k2_gather(xT, tblpack, out3, xbuf, gbuf, sbuf, ubuf, obuf2, sem):
    wid = _worker_id()
    iota = lax.iota(jnp.int32, 16)

    def bt_body(tb, carry):
        bt = wid * _BT_PER_W + tb
        pltpu.sync_copy(xT.at[pl.ds(0, 32), pl.ds(bt * 128, 128)], xbuf)

        def gh_body(f, c2):
            for k in range(8):
                iv = xbuf[f, pl.ds(k * 16, 16)]
                gbuf[f, pl.ds(k * 16, 16)] = lax.shift_right_logical(iv, 3)
                sbuf[f, pl.ds(k * 16, 16)] = lax.bitwise_and(iv, 7)
            return c2
        lax.fori_loop(0, _NF, gh_body, 0)

        def f_body(f, c2):
            pltpu.async_copy(tblpack.at[gbuf.at[f]], ubuf, sem).wait()

            for dt in range(2):
                for dr in range(8):
                    for k in range(8):
                        cvec = iota + (16 * k)
                        sv = sbuf[f, pl.ds(k * 16, 16)]
                        wvec = sv * 16 + (8 * dt + dr)
                        obuf2[dt, dr, pl.ds(16 * k, 16)] = (
                            plsc.load_gather(ubuf, [cvec, wvec]))
                pltpu.sync_copy(
                    obuf2.at[dt],
                    out3.at[f, pl.ds(8 * dt, 8), pl.ds(bt * 128, 128)])
            return c2
        lax.fori_loop(0, _NF, f_body, 0)
        return carry

    lax.fori_loop(0, _BT_PER_W, bt_body, 0)


def kernel(x, table):
    tableT = table.T
    tailT = jnp.pad(tableT[:, _CB_FULL * 128:], ((0, 0), (0, 128 - _TAIL)))
    xTp = jnp.pad(x.T, ((0, 32 - _NF), (0, 0)))
    tblpack = _k1_pack(tableT, tailT)
    out3 = _k2_gather(xTp, tblpack)
    return out3.transpose(2, 0, 1)


# speedup vs baseline: 1.7197x; 1.7197x over previous
import functools

import jax
import jax.numpy as jnp
from jax import lax
from jax.experimental import pallas as pl
from jax.experimental.pallas import tpu as pltpu
from jax.experimental.pallas import tpu_sc as plsc

_VOCAB = 38462 * 26
_D = 16
_BATCH = 16384
_NF = 26

_NC = 2
_NS = 16
_NW = _NC * _NS

_CB_FULL = _VOCAB // 128
_TAIL = _VOCAB - _CB_FULL * 128
_GROWS = _CB_FULL * 16 + 16
_TMAIN = _CB_FULL // _NW

_BT = _BATCH // 128
_BT_PER_W = _BT // _NW

_mesh = plsc.VectorSubcoreMesh(core_axis_name="c", subcore_axis_name="s")
_params = pltpu.CompilerParams(use_tc_tiling_on_sc=True,
                               needs_layout_passes=False)


def _worker_id():
    return lax.axis_index("s") * _NC + lax.axis_index("c")


@functools.partial(
    pl.kernel,
    mesh=_mesh,
    out_type=jax.ShapeDtypeStruct((_GROWS, 128), jnp.float32),
    scratch_types=[
        pltpu.VMEM((2, 16, 128), jnp.float32),
        pltpu.VMEM((2, 16, 128), jnp.float32),
        pltpu.SemaphoreType.DMA,
        pltpu.SemaphoreType.DMA,
        pltpu.SemaphoreType.DMA,
        pltpu.SemaphoreType.DMA,
    ],
    compiler_params=_params,
)
def _k1_pack(tableT, tailT, tblpack, tbuf, obuf, si0, si1, so0, so1):
    wid = _worker_id()
    iota = lax.iota(jnp.int32, 16)
    si = (si0, si1)
    so = (so0, so1)

    def in_slice(cb):
        return tableT.at[pl.ds(0, 16), pl.ds(cb * 128, 128)]

    def out_slice(cb):
        return tblpack.at[pl.ds(cb * 16, 16), :]

    def transpose_block(tsrc, odst):
        def cbody(c8, carry):
            for j in range(8):
                col = plsc.load_gather(
                    tsrc, [iota, jnp.full((16,), j, jnp.int32) + c8 * 8])
                odst[c8, pl.ds(j * 16, 16)] = col
            return carry
        lax.fori_loop(0, 16, cbody, 0)

    pltpu.async_copy(in_slice(wid), tbuf.at[0], si[0])

    def sbody(s, carry):
        for p in range(2):
            t = s * 2 + p
            cb = t * _NW + wid
            pltpu.make_async_copy(in_slice(cb), tbuf.at[p], si[p]).wait()

            @pl.when(t < _TMAIN - 1)
            def _():
                pltpu.async_copy(in_slice(cb + _NW), tbuf.at[1 - p],
                                 si[1 - p])

            @pl.when(s >= 1)
            def _():
                pltpu.make_async_copy(obuf.at[p], out_slice(cb - 2 * _NW),
                                      so[p]).wait()

            transpose_block(tbuf.at[p], obuf.at[p])
            pltpu.async_copy(obuf.at[p], out_slice(cb), so[p])
        return carry

    lax.fori_loop(0, _TMAIN // 2, sbody, 0)
    pltpu.make_async_copy(obuf.at[0], out_slice((_TMAIN - 2) * _NW + wid),
                          so[0]).wait()
    pltpu.make_async_copy(obuf.at[1], out_slice((_TMAIN - 1) * _NW + wid),
                          so[1]).wait()

    @pl.when(wid < 4)
    def _():
        cb = _TMAIN * _NW + wid
        pltpu.sync_copy(in_slice(cb), tbuf.at[0])
        transpose_block(tbuf.at[0], obuf.at[0])
        pltpu.sync_copy(obuf.at[0], out_slice(cb))

    @pl.when(wid == 4)
    def _():
        pltpu.sync_copy(tailT, tbuf.at[0])
        transpose_block(tbuf.at[0], obuf.at[0])
        pltpu.sync_copy(obuf.at[0], out_slice(_CB_FULL))


@functools.partial(
    pl.kernel,
    mesh=_mesh,
    out_type=jax.ShapeDtypeStruct((_NF, _D, _BATCH), jnp.float32),
    scratch_types=[
        pltpu.VMEM((32, 128), jnp.int32),
        pltpu.VMEM((_NF, 128), jnp.int32),
        pltpu.VMEM((_NF, 128), jnp.int32),
        pltpu.VMEM((2, 128, 128), jnp.float32),
        pltpu.VMEM((2, 2, 8, 128), jnp.float32),
        pltpu.SemaphoreType.DMA,
        pltpu.SemaphoreType.DMA,
        pltpu.SemaphoreType.DMA,
        pltpu.SemaphoreType.DMA,
    ],
    compiler_params=_params,
)
def _k2_gather(xT, tblpack, out3, xbuf, gbuf, sbuf, ubuf, obuf2,
               sg0, sg1, so0, so1):
    wid = _worker_id()
    iota = lax.iota(jnp.int32, 16)
    sg = (sg0, sg1)
    so = (so0, so1)

    def gsl(f):
        return tblpack.at[gbuf.at[f]]

    def osl(f, dt, bt):
        return out3.at[f, pl.ds(8 * dt, 8), pl.ds(bt * 128, 128)]

    def bt_body(tb, carry):
        bt = wid * _BT_PER_W + tb
        pltpu.sync_copy(xT.at[pl.ds(0, 32), pl.ds(bt * 128, 128)], xbuf)

        def gh_body(f, c2):
            for k in range(8):
                iv = xbuf[f, pl.ds(k * 16, 16)]
                gbuf[f, pl.ds(k * 16, 16)] = lax.shift_right_logical(iv, 3)
                sbuf[f, pl.ds(k * 16, 16)] = (
                    lax.shift_left(lax.bitwise_and(iv, 7), 4))
            return c2
        lax.fori_loop(0, _NF, gh_body, 0)

        pltpu.async_copy(gsl(0), ubuf.at[0], sg[0])

        def fs_body(fs, c2):
            for p in range(2):
                f = fs * 2 + p
                pltpu.make_async_copy(gsl(f), ubuf.at[p], sg[p]).wait()

                @pl.when(f < _NF - 1)
                def _():
                    pltpu.async_copy(gsl(f + 1), ubuf.at[1 - p], sg[1 - p])

                @pl.when(fs >= 1)
                def _():
                    for dt in range(2):
                        pltpu.make_async_copy(
                            obuf2.at[p, dt], osl(f - 2, dt, bt),
                            so[p]).wait()

                for k in range(8):
                    cvec = iota + (16 * k)
                    sv = sbuf[f, pl.ds(k * 16, 16)]
                    for dt in range(2):
                        for dr in range(8):
                            obuf2[p, dt, dr, pl.ds(16 * k, 16)] = (
                                plsc.load_gather(
                                    ubuf.at[p], [cvec, sv + (8 * dt + dr)]))
                for dt in range(2):
                    pltpu.async_copy(obuf2.at[p, dt], osl(f, dt, bt), so[p])
            return c2
        lax.fori_loop(0, _NF // 2, fs_body, 0)

        for p in range(2):
            for dt in range(2):
                pltpu.make_async_copy(
                    obuf2.at[p, dt], osl(_NF - 2 + p, dt, bt), so[p]).wait()
        return carry

    lax.fori_loop(0, _BT_PER_W, bt_body, 0)


def kernel(x, table):
    tableT = table.T
    tailT = jnp.pad(tableT[:, _CB_FULL * 128:], ((0, 0), (0, 128 - _TAIL)))
    xTp = jnp.pad(x.T, ((0, 32 - _NF), (0, 0)))
    tblpack = _k1_pack(tableT, tailT)
    out3 = _k2_gather(xTp, tblpack)
    return out3.transpose(2, 0, 1)
